# baseline (speedup 1.0000x reference)
import jax
import jax.numpy as jnp
from jax.experimental import pallas as pl

_BB = 128
_P = 10000


def _mask_kernel(n_ref, out_ref):
    thresh = n_ref[0] * 100
    cols = jax.lax.broadcasted_iota(jnp.int32, (_BB, _P), 1)
    out_ref[...] = (cols < thresh).astype(jnp.float32)


def kernel(tensor_span):
    b = tensor_span.shape[0]
    n = tensor_span[:, 0]
    nb = b // _BB
    n3 = n.reshape(nb, _BB, 1)
    out = pl.pallas_call(
        _mask_kernel,
        grid=(nb,),
        in_specs=[pl.BlockSpec((1, _BB, 1), lambda i: (i, 0, 0))],
        out_specs=pl.BlockSpec((_BB, _P), lambda i: (i, 0)),
        out_shape=jax.ShapeDtypeStruct((b, _P), jnp.float32),
    )(n3)
    return out.reshape(b, 100, 100)


# speedup vs baseline: 1.3826x; 1.3826x over previous
import jax
import jax.numpy as jnp
from jax.experimental import pallas as pl
from jax.experimental.pallas import tpu as pltpu

_B = 8192
_BB = 64
_K = 8
_NC = _B // _BB


def _mask_kernel(n_ref, out_ref, buf, sem):
    rows = jax.lax.broadcasted_iota(jnp.int32, (100, 100), 0)

    def body(c, carry):
        k = jax.lax.rem(c, _K)

        @pl.when(c >= _K)
        def _wait_prev():
            pltpu.make_async_copy(
                buf.at[k], out_ref.at[pl.ds((c - _K) * _BB, _BB)], sem.at[k]
            ).wait()

        bk = buf.at[k]
        for j in range(_BB):
            bk[j] = (rows < n_ref[c * _BB + j]).astype(jnp.float32)

        pltpu.make_async_copy(
            buf.at[k], out_ref.at[pl.ds(c * _BB, _BB)], sem.at[k]
        ).start()
        return carry

    jax.lax.fori_loop(0, _NC, body, 0)

    def drain(i, carry):
        c = _NC - _K + i
        k = jax.lax.rem(c, _K)
        pltpu.make_async_copy(
            buf.at[k], out_ref.at[pl.ds(c * _BB, _BB)], sem.at[k]
        ).wait()
        return carry

    jax.lax.fori_loop(0, _K, drain, 0)


def kernel(tensor_span):
    b = tensor_span.shape[0]
    n = tensor_span[:, 0]
    return pl.pallas_call(
        _mask_kernel,
        in_specs=[pl.BlockSpec(memory_space=pltpu.MemorySpace.SMEM)],
        out_specs=pl.BlockSpec(memory_space=pl.ANY),
        out_shape=jax.ShapeDtypeStruct((b, 100, 100), jnp.float32),
        scratch_shapes=[
            pltpu.VMEM((_K, _BB, 100, 100), jnp.float32),
            pltpu.SemaphoreType.DMA((_K,)),
        ],
    )(n)
